# baseline (speedup 1.0000x reference)
import functools

import jax
import jax.numpy as jnp
from jax import lax
from jax.experimental import pallas as pl
from jax.experimental.pallas import tpu as pltpu
from jax.experimental.pallas import tpu_sc as plsc

_V = 1000
_D = 1000
_NSMP = 4096
_T = 20
_B = _NSMP * _T
_NC, _NS = 2, 16
_NW = _NC * _NS
_SPW = _NSMP // _NW
_S = 2
_R = _S * _T
_C = _SPW // _S


def _sigmoid_body(x_ref, o_ref):
    x = x_ref[...]
    o_ref[...] = 1.0 / (1.0 + jnp.exp(-x))


def _sigmoid_table(logits):
    return pl.pallas_call(
        _sigmoid_body,
        out_shape=jax.ShapeDtypeStruct(logits.shape, logits.dtype),
    )(logits)


_mesh = plsc.VectorSubcoreMesh(core_axis_name="c", subcore_axis_name="s")


@functools.partial(
    pl.kernel,
    out_type=jax.ShapeDtypeStruct((_NSMP, _T, _D), jnp.float32),
    mesh=_mesh,
    scratch_types=[
        pltpu.VMEM((_NW * _SPW * _T // _NW,), jnp.int32),
        pltpu.VMEM((_R, _D), jnp.float32),
        pltpu.VMEM((_R, _D), jnp.float32),
        pltpu.SemaphoreType.DMA,
        pltpu.SemaphoreType.DMA,
        pltpu.SemaphoreType.DMA,
        pltpu.SemaphoreType.DMA,
    ],
    compiler_params=pltpu.CompilerParams(use_tc_tiling_on_sc=False),
)
def _gather_kernel(table_hbm, idx_hbm, out_hbm, idx_v, g0, g1,
                   gs0, gs1, os0, os1):
    sid = lax.axis_index("s")
    wid = sid * _NC + lax.axis_index("c")
    ibase = wid * _SPW * _T
    sbase = wid * _SPW
    pltpu.sync_copy(idx_hbm.at[pl.ds(ibase, _SPW * _T)], idx_v)

    bufs = (g0, g1)
    gsems = (gs0, gs1)
    osems = (os0, os1)

    def gather_start(c, b):
        pltpu.make_async_copy(
            table_hbm.at[idx_v.at[pl.ds(c * _R, _R)]], bufs[b], gsems[b]
        ).start()

    def gather_wait(b):
        pltpu.make_async_copy(
            table_hbm.at[idx_v.at[pl.ds(0, _R)]], bufs[b], gsems[b]
        ).wait()

    def out_start(c, b):
        for s in range(_S):
            pltpu.make_async_copy(
                bufs[b].at[pl.ds(s * _T, _T)],
                out_hbm.at[sbase + c * _S + s],
                osems[b],
            ).start()

    def out_wait(b):
        for s in range(_S):
            pltpu.make_async_copy(
                bufs[b].at[pl.ds(s * _T, _T)], out_hbm.at[sbase + s], osems[b]
            ).wait()

    gather_start(0, 0)
    gather_start(1, 1)

    @pl.loop(0, _C, step=2)
    def _(c0):
        for b in range(2):
            c = c0 + b
            gather_wait(b)
            out_start(c, b)
            out_wait(b)

            @pl.when(c + 2 < _C)
            def _():
                gather_start(c + 2, b)


def kernel(idx, morphisms_logits):
    table = _sigmoid_table(morphisms_logits)
    idx_flat = idx.reshape(-1).astype(jnp.int32)
    return _gather_kernel(table, idx_flat)


# speedup vs baseline: 1.0464x; 1.0464x over previous
import functools

import jax
import jax.numpy as jnp
from jax import lax
from jax.experimental import pallas as pl
from jax.experimental.pallas import tpu as pltpu
from jax.experimental.pallas import tpu_sc as plsc

_V = 1000
_D = 1000
_DP = 1024
_NSMP = 4096
_T = 20
_NC, _NS = 2, 16
_NW = _NC * _NS
_NDB = 8
_NBB = 4
_BB = _NSMP // _NBB
_BH = _BB // 2
_RC = 32


def _sigmoid_t_body(x_ref, o_ref):
    x = x_ref[...]
    o_ref[...] = (1.0 / (1.0 + jnp.exp(-x))).T


def _sigmoid_t_table(logits_padded):
    return pl.pallas_call(
        _sigmoid_t_body,
        grid=(8, 8),
        in_specs=[pl.BlockSpec((128, 128), lambda i, j: (j, i))],
        out_specs=pl.BlockSpec((128, 128), lambda i, j: (i, j)),
        out_shape=jax.ShapeDtypeStruct((_DP, _DP), jnp.float32),
    )(logits_padded)


_mesh = plsc.VectorSubcoreMesh(core_axis_name="c", subcore_axis_name="s")


@functools.partial(
    pl.kernel,
    out_type=jax.ShapeDtypeStruct((_T, _D, _NSMP), jnp.float32),
    mesh=_mesh,
    scratch_types=[
        pltpu.VMEM((_RC * _DP,), jnp.float32),
        pltpu.VMEM((_T * _BB,), jnp.int32),
        pltpu.VMEM((_RC, _BH), jnp.float32),
        pltpu.VMEM((_RC, _BH), jnp.float32),
        pltpu.SemaphoreType.DMA,
        pltpu.SemaphoreType.DMA,
    ],
    compiler_params=pltpu.CompilerParams(needs_layout_passes=False),
)
def _tgather_kernel(tt_hbm, idxt_hbm, out_hbm, ttbuf, idxbuf, ob0, ob1,
                    os0, os1):
    wid = lax.axis_index("s") * _NC + lax.axis_index("c")
    dblk = wid // _NBB
    dbase = dblk * 128
    b0 = (wid % _NBB) * _BB

    @pl.loop(0, _T)
    def _(t):
        pltpu.sync_copy(
            idxt_hbm.at[pl.ds(t * _NSMP + b0, _BB)],
            idxbuf.at[pl.ds(t * _BB, _BB)],
        )

    obufs = (ob0, ob1)
    osems = (os0, os1)

    def out_wait(h, rows):
        pltpu.make_async_copy(
            obufs[h].at[pl.ds(0, rows)],
            out_hbm.at[0, pl.ds(0, rows), pl.ds(0, _BH)],
            osems[h],
        ).wait()

    def do_chunk(d0, rows):
        pltpu.sync_copy(
            tt_hbm.at[pl.ds(d0 * _DP, rows * _DP)],
            ttbuf.at[pl.ds(0, rows * _DP)],
        )

        @pl.loop(0, 2 * _T, step=2)
        def _(k0):
            for h in range(2):
                k = k0 + h
                t = k // 2
                bh = b0 + (k % 2) * _BH
                ob = obufs[h]

                @pl.when(k0 >= 2)
                def _():
                    out_wait(h, rows)

                @pl.loop(0, _BH // 16)
                def _(g):
                    iv = idxbuf[pl.ds(t * _BB + (k % 2) * _BH + g * 16, 16)]
                    for d in range(rows):
                        vals = plsc.load_gather(ttbuf, [iv + d * _DP])
                        ob[d, pl.ds(g * 16, 16)] = vals

                pltpu.make_async_copy(
                    ob.at[pl.ds(0, rows)],
                    out_hbm.at[t, pl.ds(d0, rows), pl.ds(bh, _BH)],
                    osems[h],
                ).start()

        out_wait(0, rows)
        out_wait(1, rows)

    @pl.loop(0, 3)
    def _(c):
        do_chunk(dbase + c * _RC, _RC)

    @pl.when(dblk < _NDB - 1)
    def _():
        do_chunk(dbase + 3 * _RC, _RC)

    @pl.when(dblk == _NDB - 1)
    def _():
        do_chunk(dbase + 3 * _RC, 8)


def kernel(idx, morphisms_logits):
    logits_p = jnp.pad(morphisms_logits, ((0, _DP - _V), (0, _DP - _D)))
    tt_flat = _sigmoid_t_table(logits_p).reshape(-1)
    idxt_flat = jnp.transpose(idx.astype(jnp.int32)).reshape(-1)
    out_t = _tgather_kernel(tt_flat, idxt_flat)
    return jnp.transpose(out_t, (2, 0, 1))
